# baseline (speedup 1.0000x reference)
import jax
import jax.numpy as jnp
from jax.experimental import pallas as pl
from jax.experimental.pallas import tpu as pltpu

B = 4
TOP_SEQ = 8
TOP_POS = 5
TOPK = 3
NUM_WORDS = 1000
N_CLUSTERS = 4096
DIM_V = 256
C_FM = 768
NPAIR = B * TOP_SEQ
NPW = NPAIR * TOP_POS
CBLK = 512
NBLK = N_CLUSTERS // CBLK
NEG = float("-inf")
BIG = 1 << 30


def _iota(shape, dim):
    return jax.lax.broadcasted_iota(jnp.int32, shape, dim)


def _fused_kernel(pred_ref, fs_t_ref, xfm_ref, fmax_ref, fmin_ref,
                  cw_ref, cli_ref, vcc_ref,
                  dwords_ref, dcls_ref, fv_ref,
                  vals_s, words_s, onehot_s, hv_s,
                  mv_s, mi_s, pv_s, pi_s, sim_s):
    i = pl.program_id(0)

    @pl.when(i == 0)
    def _prep():
        p = pred_ref[:, :]
        col = _iota((NPAIR, NUM_WORDS), 1)
        vals = []
        words = []
        for _ in range(TOP_POS):
            mx = jnp.max(p, axis=1, keepdims=True)
            idx = jnp.min(jnp.where(p == mx, col, BIG), axis=1, keepdims=True)
            vals.append(mx)
            words.append(idx)
            p = jnp.where(col == idx, NEG, p)
        vals_s[:, :] = jnp.concatenate(vals, axis=1)
        words_s[:, :] = jnp.concatenate(words, axis=1)
        for w in range(TOP_POS):
            onehot_s[w * NPAIR:(w + 1) * NPAIR, :] = (
                col == words[w]).astype(jnp.bfloat16)
        fmax = fmax_ref[0, 0]
        fmin = fmin_ref[0, 0]
        scaled = (fs_t_ref[:, :] - fmin) / (fmax - fmin)
        hv_s[:, :] = jnp.concatenate([scaled, 1.0 - scaled], axis=0)
        fv_ref[:, :] = jnp.mean(xfm_ref[:, :], axis=1, keepdims=True)
        neg = jnp.full((NPAIR, TOP_POS), NEG, dtype=jnp.float32)
        zero = jnp.zeros((NPAIR, TOP_POS), dtype=jnp.int32)
        mv_s[:, :] = neg
        pv_s[:, :] = neg
        mi_s[:, :] = zero
        pi_s[:, :] = zero

    ind = jax.lax.dot_general(
        onehot_s[:, :], cli_ref[:, :],
        dimension_numbers=(((1,), (1,)), ((), ())),
        preferred_element_type=jnp.float32)
    hv = hv_s[:, :]
    cw = jnp.transpose(cw_ref[:, :])
    s_cw = jnp.sum(cw, axis=0, keepdims=True)
    s_hv = jnp.sum(hv, axis=0, keepdims=True)

    for pair in range(NPAIR):
        hvp = jax.lax.slice(hv, (0, pair), (2 * DIM_V, pair + 1))
        shvp = jax.lax.slice(s_hv, (0, pair), (1, pair + 1))
        inter = jnp.sum(jnp.minimum(hvp, cw), axis=0, keepdims=True)
        sim_s[pair:pair + 1, :] = inter / (shvp + s_cw - inter)
    sim = sim_s[:, :]

    colb = _iota((NPAIR, CBLK), 1) + i * CBLK
    pm = jnp.max(sim, axis=1, keepdims=True)
    pi = jnp.min(jnp.where(sim == pm, colb, BIG), axis=1, keepdims=True)
    for w in range(TOP_POS):
        ind_w = ind[w * NPAIR:(w + 1) * NPAIR, :]
        masked = jnp.where(ind_w > 0.0, sim, NEG)
        bm = jnp.max(masked, axis=1, keepdims=True)
        bi = jnp.min(jnp.where(masked == bm, colb, BIG), axis=1, keepdims=True)
        upd = bm > mv_s[:, w:w + 1]
        mv_s[:, w:w + 1] = jnp.where(upd, bm, mv_s[:, w:w + 1])
        mi_s[:, w:w + 1] = jnp.where(upd, bi, mi_s[:, w:w + 1])
        updp = pm > pv_s[:, w:w + 1]
        pv_s[:, w:w + 1] = jnp.where(updp, pm, pv_s[:, w:w + 1])
        pi_s[:, w:w + 1] = jnp.where(updp, pi, pi_s[:, w:w + 1])

    @pl.when(i == NBLK - 1)
    def _final():
        has_valid = mv_s[:, :] != NEG
        sims5 = jnp.where(has_valid, mv_s[:, :], pv_s[:, :])
        sel5 = jnp.where(has_valid, mi_s[:, :], pi_s[:, :])
        vcc = vcc_ref[:, :]
        colc = _iota((NPAIR, N_CLUSTERS), 1)
        lbls = []
        for w in range(TOP_POS):
            lbls.append(jnp.sum(
                jnp.where(colc == sel5[:, w:w + 1], vcc, 0),
                axis=1, keepdims=True))
        lbls5 = jnp.concatenate(lbls, axis=1)
        fwp = sims5 * vals_s[:, :]
        col5 = _iota((NPAIR, TOP_POS), 1)
        cmax = jnp.max(fwp, axis=1, keepdims=True)
        widx = jnp.min(jnp.where(fwp == cmax, col5, BIG), axis=1,
                       keepdims=True)
        dwords_ref[:, :] = jnp.sum(
            jnp.where(col5 == widx, words_s[:, :], 0), axis=1, keepdims=True)
        clabels = jnp.sum(jnp.where(col5 == widx, lbls5, 0), axis=1,
                          keepdims=True)
        rio = _iota((NPAIR, B), 0)
        bio = _iota((NPAIR, B), 1)
        cs = jnp.where((rio // TOP_SEQ) == bio, cmax, NEG)
        for r in range(TOPK):
            m = jnp.max(cs, axis=0, keepdims=True)
            sel = jnp.min(jnp.where(cs == m, rio, BIG), axis=0, keepdims=True)
            dcls_ref[r:r + 1, :] = jnp.sum(
                jnp.where(rio == sel, clabels, 0), axis=0, keepdims=True)
            cs = jnp.where(rio == sel, NEG, cs)


def kernel(predicts_t, x_fm, feature_s, cluster_weights,
           cluster_label_indicators, valid_cluster_class,
           feature_max, feature_min):
    cli = cluster_label_indicators.astype(jnp.bfloat16)
    pred_r = predicts_t.reshape(NPAIR, NUM_WORDS)
    fs_t = jnp.transpose(feature_s, (1, 0, 2)).reshape(NPAIR, DIM_V).T
    xfm_r = x_fm.reshape(B * C_FM, 14 * 14)
    vcc = valid_cluster_class.astype(jnp.int32).reshape(1, N_CLUSTERS)
    fmax = feature_max.astype(jnp.float32).reshape(1, 1)
    fmin = feature_min.astype(jnp.float32).reshape(1, 1)

    f32 = jnp.float32
    dwords, dcls, fv = pl.pallas_call(
        _fused_kernel,
        grid=(NBLK,),
        in_specs=[
            pl.BlockSpec((NPAIR, NUM_WORDS), lambda i: (0, 0)),
            pl.BlockSpec((DIM_V, NPAIR), lambda i: (0, 0)),
            pl.BlockSpec((B * C_FM, 14 * 14), lambda i: (0, 0)),
            pl.BlockSpec((1, 1), lambda i: (0, 0)),
            pl.BlockSpec((1, 1), lambda i: (0, 0)),
            pl.BlockSpec((CBLK, 2 * DIM_V), lambda i: (i, 0)),
            pl.BlockSpec((CBLK, NUM_WORDS), lambda i: (i, 0)),
            pl.BlockSpec((1, N_CLUSTERS), lambda i: (0, 0)),
        ],
        out_specs=(
            pl.BlockSpec((NPAIR, 1), lambda i: (0, 0)),
            pl.BlockSpec((TOPK, B), lambda i: (0, 0)),
            pl.BlockSpec((B * C_FM, 1), lambda i: (0, 0)),
        ),
        out_shape=(
            jax.ShapeDtypeStruct((NPAIR, 1), jnp.int32),
            jax.ShapeDtypeStruct((TOPK, B), jnp.int32),
            jax.ShapeDtypeStruct((B * C_FM, 1), f32),
        ),
        scratch_shapes=[
            pltpu.VMEM((NPAIR, TOP_POS), f32),
            pltpu.VMEM((NPAIR, TOP_POS), jnp.int32),
            pltpu.VMEM((NPW, NUM_WORDS), jnp.bfloat16),
            pltpu.VMEM((2 * DIM_V, NPAIR), f32),
            pltpu.VMEM((NPAIR, TOP_POS), f32),
            pltpu.VMEM((NPAIR, TOP_POS), jnp.int32),
            pltpu.VMEM((NPAIR, TOP_POS), f32),
            pltpu.VMEM((NPAIR, TOP_POS), jnp.int32),
            pltpu.VMEM((NPAIR, CBLK), f32),
        ],
    )(pred_r, fs_t, xfm_r, fmax, fmin, cluster_weights, cli, vcc)

    decision_words = dwords.reshape(B, TOP_SEQ)
    decision_classes_topk = dcls.T
    feature_v = fv.reshape(B, C_FM)
    return (predicts_t, feature_v, decision_words, decision_classes_topk)


# speedup vs baseline: 1.0418x; 1.0418x over previous
import jax
import jax.numpy as jnp
from jax.experimental import pallas as pl
from jax.experimental.pallas import tpu as pltpu

B = 4
TOP_SEQ = 8
TOP_POS = 5
TOPK = 3
NUM_WORDS = 1000
N_CLUSTERS = 4096
DIM_V = 256
C_FM = 768
NPAIR = B * TOP_SEQ
NPW = NPAIR * TOP_POS
CBLK = 1024
NBLK = N_CLUSTERS // CBLK
NEG = float("-inf")
BIG = 1 << 30


def _iota(shape, dim):
    return jax.lax.broadcasted_iota(jnp.int32, shape, dim)


def _fused_kernel(pred_ref, fs_t_ref, xfm_ref, fmax_ref, fmin_ref,
                  cw_ref, cli_ref, vcc_ref,
                  dwords_ref, dcls_ref, fv_ref,
                  vals_s, words_s, onehot_s, hv_s,
                  mv_s, mi_s, pv_s, pi_s, sim_s):
    i = pl.program_id(0)

    @pl.when(i == 0)
    def _prep():
        p = pred_ref[:, :]
        col = _iota((NPAIR, NUM_WORDS), 1)
        vals = []
        words = []
        for _ in range(TOP_POS):
            mx = jnp.max(p, axis=1, keepdims=True)
            idx = jnp.min(jnp.where(p == mx, col, BIG), axis=1, keepdims=True)
            vals.append(mx)
            words.append(idx)
            p = jnp.where(col == idx, NEG, p)
        vals_s[:, :] = jnp.concatenate(vals, axis=1)
        words_s[:, :] = jnp.concatenate(words, axis=1)
        for w in range(TOP_POS):
            onehot_s[w * NPAIR:(w + 1) * NPAIR, :] = (
                col == words[w]).astype(jnp.float32)
        fmax = fmax_ref[0, 0]
        fmin = fmin_ref[0, 0]
        scaled = (fs_t_ref[:, :] - fmin) / (fmax - fmin)
        hv_s[:, :] = jnp.concatenate([scaled, 1.0 - scaled], axis=0)
        fv_ref[:, :] = jnp.mean(xfm_ref[:, :], axis=1, keepdims=True)
        neg = jnp.full((NPAIR, TOP_POS), NEG, dtype=jnp.float32)
        zero = jnp.zeros((NPAIR, TOP_POS), dtype=jnp.int32)
        mv_s[:, :] = neg
        pv_s[:, :] = neg
        mi_s[:, :] = zero
        pi_s[:, :] = zero

    ind = jax.lax.dot_general(
        onehot_s[:, :], cli_ref[:, :],
        dimension_numbers=(((1,), (1,)), ((), ())),
        preferred_element_type=jnp.float32)
    hv = hv_s[:, :]
    cw = jnp.transpose(cw_ref[:, :])
    s_cw = jnp.sum(cw, axis=0, keepdims=True)
    s_hv = jnp.sum(hv, axis=0, keepdims=True)

    for pair in range(NPAIR):
        hvp = jax.lax.slice(hv, (0, pair), (2 * DIM_V, pair + 1))
        shvp = jax.lax.slice(s_hv, (0, pair), (1, pair + 1))
        inter = jnp.sum(jnp.minimum(hvp, cw), axis=0, keepdims=True)
        sim_s[pair:pair + 1, :] = inter / (shvp + s_cw - inter)
    sim = sim_s[:, :]

    colb = _iota((NPAIR, CBLK), 1) + i * CBLK
    pm = jnp.max(sim, axis=1, keepdims=True)
    pi = jnp.min(jnp.where(sim == pm, colb, BIG), axis=1, keepdims=True)
    for w in range(TOP_POS):
        ind_w = ind[w * NPAIR:(w + 1) * NPAIR, :]
        masked = jnp.where(ind_w > 0.0, sim, NEG)
        bm = jnp.max(masked, axis=1, keepdims=True)
        bi = jnp.min(jnp.where(masked == bm, colb, BIG), axis=1, keepdims=True)
        upd = bm > mv_s[:, w:w + 1]
        mv_s[:, w:w + 1] = jnp.where(upd, bm, mv_s[:, w:w + 1])
        mi_s[:, w:w + 1] = jnp.where(upd, bi, mi_s[:, w:w + 1])
        updp = pm > pv_s[:, w:w + 1]
        pv_s[:, w:w + 1] = jnp.where(updp, pm, pv_s[:, w:w + 1])
        pi_s[:, w:w + 1] = jnp.where(updp, pi, pi_s[:, w:w + 1])

    @pl.when(i == NBLK - 1)
    def _final():
        has_valid = mv_s[:, :] != NEG
        sims5 = jnp.where(has_valid, mv_s[:, :], pv_s[:, :])
        sel5 = jnp.where(has_valid, mi_s[:, :], pi_s[:, :])
        vcc = vcc_ref[:, :]
        colc = _iota((NPAIR, N_CLUSTERS), 1)
        lbls = []
        for w in range(TOP_POS):
            lbls.append(jnp.sum(
                jnp.where(colc == sel5[:, w:w + 1], vcc, 0),
                axis=1, keepdims=True))
        lbls5 = jnp.concatenate(lbls, axis=1)
        fwp = sims5 * vals_s[:, :]
        col5 = _iota((NPAIR, TOP_POS), 1)
        cmax = jnp.max(fwp, axis=1, keepdims=True)
        widx = jnp.min(jnp.where(fwp == cmax, col5, BIG), axis=1,
                       keepdims=True)
        dwords_ref[:, :] = jnp.sum(
            jnp.where(col5 == widx, words_s[:, :], 0), axis=1, keepdims=True)
        clabels = jnp.sum(jnp.where(col5 == widx, lbls5, 0), axis=1,
                          keepdims=True)
        rio = _iota((NPAIR, B), 0)
        bio = _iota((NPAIR, B), 1)
        cs = jnp.where((rio // TOP_SEQ) == bio, cmax, NEG)
        for r in range(TOPK):
            m = jnp.max(cs, axis=0, keepdims=True)
            sel = jnp.min(jnp.where(cs == m, rio, BIG), axis=0, keepdims=True)
            dcls_ref[r:r + 1, :] = jnp.sum(
                jnp.where(rio == sel, clabels, 0), axis=0, keepdims=True)
            cs = jnp.where(rio == sel, NEG, cs)


def kernel(predicts_t, x_fm, feature_s, cluster_weights,
           cluster_label_indicators, valid_cluster_class,
           feature_max, feature_min):
    cli = cluster_label_indicators.astype(jnp.float32)
    pred_r = predicts_t.reshape(NPAIR, NUM_WORDS)
    fs_t = jnp.transpose(feature_s, (1, 0, 2)).reshape(NPAIR, DIM_V).T
    xfm_r = x_fm.reshape(B * C_FM, 14 * 14)
    vcc = valid_cluster_class.astype(jnp.int32).reshape(1, N_CLUSTERS)
    fmax = feature_max.astype(jnp.float32).reshape(1, 1)
    fmin = feature_min.astype(jnp.float32).reshape(1, 1)

    f32 = jnp.float32
    dwords, dcls, fv = pl.pallas_call(
        _fused_kernel,
        grid=(NBLK,),
        in_specs=[
            pl.BlockSpec((NPAIR, NUM_WORDS), lambda i: (0, 0)),
            pl.BlockSpec((DIM_V, NPAIR), lambda i: (0, 0)),
            pl.BlockSpec((B * C_FM, 14 * 14), lambda i: (0, 0)),
            pl.BlockSpec((1, 1), lambda i: (0, 0)),
            pl.BlockSpec((1, 1), lambda i: (0, 0)),
            pl.BlockSpec((CBLK, 2 * DIM_V), lambda i: (i, 0)),
            pl.BlockSpec((CBLK, NUM_WORDS), lambda i: (i, 0)),
            pl.BlockSpec((1, N_CLUSTERS), lambda i: (0, 0)),
        ],
        out_specs=(
            pl.BlockSpec((NPAIR, 1), lambda i: (0, 0)),
            pl.BlockSpec((TOPK, B), lambda i: (0, 0)),
            pl.BlockSpec((B * C_FM, 1), lambda i: (0, 0)),
        ),
        out_shape=(
            jax.ShapeDtypeStruct((NPAIR, 1), jnp.int32),
            jax.ShapeDtypeStruct((TOPK, B), jnp.int32),
            jax.ShapeDtypeStruct((B * C_FM, 1), f32),
        ),
        scratch_shapes=[
            pltpu.VMEM((NPAIR, TOP_POS), f32),
            pltpu.VMEM((NPAIR, TOP_POS), jnp.int32),
            pltpu.VMEM((NPW, NUM_WORDS), f32),
            pltpu.VMEM((2 * DIM_V, NPAIR), f32),
            pltpu.VMEM((NPAIR, TOP_POS), f32),
            pltpu.VMEM((NPAIR, TOP_POS), jnp.int32),
            pltpu.VMEM((NPAIR, TOP_POS), f32),
            pltpu.VMEM((NPAIR, TOP_POS), jnp.int32),
            pltpu.VMEM((NPAIR, CBLK), f32),
        ],
    )(pred_r, fs_t, xfm_r, fmax, fmin, cluster_weights, cli, vcc)

    decision_words = dwords.reshape(B, TOP_SEQ)
    decision_classes_topk = dcls.T
    feature_v = fv.reshape(B, C_FM)
    return (predicts_t, feature_v, decision_words, decision_classes_topk)


# speedup vs baseline: 1.0506x; 1.0085x over previous
import jax
import jax.numpy as jnp
from jax.experimental import pallas as pl
from jax.experimental.pallas import tpu as pltpu

B = 4
TOP_SEQ = 8
TOP_POS = 5
TOPK = 3
NUM_WORDS = 1000
N_CLUSTERS = 4096
DIM_V = 256
C_FM = 768
NPAIR = B * TOP_SEQ
NPW = NPAIR * TOP_POS
CBLK = 512
NBLK = N_CLUSTERS // CBLK
NEG = float("-inf")
BIG = 1 << 30


def _iota(shape, dim):
    return jax.lax.broadcasted_iota(jnp.int32, shape, dim)


def _fused_kernel(pred_ref, fs_t_ref, xfm_ref, fmax_ref, fmin_ref,
                  cw_ref, cli_ref, vcc_ref,
                  dwords_ref, dcls_ref, fv_ref,
                  vals_s, words_s, onehot_s, hv_s,
                  mv_s, mi_s, pv_s, pi_s, sim_s):
    i = pl.program_id(0)

    @pl.when(i == 0)
    def _prep():
        p = pred_ref[:, :]
        col = _iota((NPAIR, NUM_WORDS), 1)
        vals = []
        words = []
        for _ in range(TOP_POS):
            mx = jnp.max(p, axis=1, keepdims=True)
            idx = jnp.min(jnp.where(p == mx, col, BIG), axis=1, keepdims=True)
            vals.append(mx)
            words.append(idx)
            p = jnp.where(col == idx, NEG, p)
        vals_s[:, :] = jnp.concatenate(vals, axis=1)
        words_s[:, :] = jnp.concatenate(words, axis=1)
        for w in range(TOP_POS):
            onehot_s[w * NPAIR:(w + 1) * NPAIR, :] = (
                col == words[w]).astype(jnp.float32)
        fmax = fmax_ref[0, 0]
        fmin = fmin_ref[0, 0]
        scaled = (fs_t_ref[:, :] - fmin) / (fmax - fmin)
        hv_s[:, :] = jnp.concatenate([scaled, 1.0 - scaled], axis=0)
        fv_ref[:, :] = jnp.mean(xfm_ref[:, :], axis=1, keepdims=True)
        neg = jnp.full((NPAIR, TOP_POS), NEG, dtype=jnp.float32)
        zero = jnp.zeros((NPAIR, TOP_POS), dtype=jnp.int32)
        mv_s[:, :] = neg
        pv_s[:, :] = neg
        mi_s[:, :] = zero
        pi_s[:, :] = zero

    ind = jax.lax.dot_general(
        onehot_s[:, :], cli_ref[:, :],
        dimension_numbers=(((1,), (1,)), ((), ())),
        preferred_element_type=jnp.float32)
    hv = hv_s[:, :]
    cw = jnp.transpose(cw_ref[:, :])
    s_cw = jnp.sum(cw, axis=0, keepdims=True)
    s_hv = jnp.sum(hv, axis=0, keepdims=True)

    for pair in range(NPAIR):
        hvp = jax.lax.slice(hv, (0, pair), (2 * DIM_V, pair + 1))
        shvp = jax.lax.slice(s_hv, (0, pair), (1, pair + 1))
        inter = jnp.sum(jnp.minimum(hvp, cw), axis=0, keepdims=True)
        sim_s[pair:pair + 1, :] = inter / (shvp + s_cw - inter)
    sim = sim_s[:, :]

    colb = _iota((NPAIR, CBLK), 1) + i * CBLK
    pm = jnp.max(sim, axis=1, keepdims=True)
    pi = jnp.min(jnp.where(sim == pm, colb, BIG), axis=1, keepdims=True)
    for w in range(TOP_POS):
        ind_w = ind[w * NPAIR:(w + 1) * NPAIR, :]
        masked = jnp.where(ind_w > 0.0, sim, NEG)
        bm = jnp.max(masked, axis=1, keepdims=True)
        bi = jnp.min(jnp.where(masked == bm, colb, BIG), axis=1, keepdims=True)
        upd = bm > mv_s[:, w:w + 1]
        mv_s[:, w:w + 1] = jnp.where(upd, bm, mv_s[:, w:w + 1])
        mi_s[:, w:w + 1] = jnp.where(upd, bi, mi_s[:, w:w + 1])
        updp = pm > pv_s[:, w:w + 1]
        pv_s[:, w:w + 1] = jnp.where(updp, pm, pv_s[:, w:w + 1])
        pi_s[:, w:w + 1] = jnp.where(updp, pi, pi_s[:, w:w + 1])

    @pl.when(i == NBLK - 1)
    def _final():
        has_valid = mv_s[:, :] != NEG
        sims5 = jnp.where(has_valid, mv_s[:, :], pv_s[:, :])
        sel5 = jnp.where(has_valid, mi_s[:, :], pi_s[:, :])
        vcc = vcc_ref[:, :]
        colc = _iota((NPAIR, N_CLUSTERS), 1)
        lbls = []
        for w in range(TOP_POS):
            lbls.append(jnp.sum(
                jnp.where(colc == sel5[:, w:w + 1], vcc, 0),
                axis=1, keepdims=True))
        lbls5 = jnp.concatenate(lbls, axis=1)
        fwp = sims5 * vals_s[:, :]
        col5 = _iota((NPAIR, TOP_POS), 1)
        cmax = jnp.max(fwp, axis=1, keepdims=True)
        widx = jnp.min(jnp.where(fwp == cmax, col5, BIG), axis=1,
                       keepdims=True)
        dwords_ref[:, :] = jnp.sum(
            jnp.where(col5 == widx, words_s[:, :], 0), axis=1, keepdims=True)
        clabels = jnp.sum(jnp.where(col5 == widx, lbls5, 0), axis=1,
                          keepdims=True)
        rio = _iota((NPAIR, B), 0)
        bio = _iota((NPAIR, B), 1)
        cs = jnp.where((rio // TOP_SEQ) == bio, cmax, NEG)
        for r in range(TOPK):
            m = jnp.max(cs, axis=0, keepdims=True)
            sel = jnp.min(jnp.where(cs == m, rio, BIG), axis=0, keepdims=True)
            dcls_ref[r:r + 1, :] = jnp.sum(
                jnp.where(rio == sel, clabels, 0), axis=0, keepdims=True)
            cs = jnp.where(rio == sel, NEG, cs)


def kernel(predicts_t, x_fm, feature_s, cluster_weights,
           cluster_label_indicators, valid_cluster_class,
           feature_max, feature_min):
    cli = cluster_label_indicators.astype(jnp.float32)
    pred_r = predicts_t.reshape(NPAIR, NUM_WORDS)
    fs_t = jnp.transpose(feature_s, (1, 0, 2)).reshape(NPAIR, DIM_V).T
    xfm_r = x_fm.reshape(B * C_FM, 14 * 14)
    vcc = valid_cluster_class.astype(jnp.int32).reshape(1, N_CLUSTERS)
    fmax = feature_max.astype(jnp.float32).reshape(1, 1)
    fmin = feature_min.astype(jnp.float32).reshape(1, 1)

    f32 = jnp.float32
    dwords, dcls, fv = pl.pallas_call(
        _fused_kernel,
        grid=(NBLK,),
        in_specs=[
            pl.BlockSpec((NPAIR, NUM_WORDS), lambda i: (0, 0)),
            pl.BlockSpec((DIM_V, NPAIR), lambda i: (0, 0)),
            pl.BlockSpec((B * C_FM, 14 * 14), lambda i: (0, 0)),
            pl.BlockSpec((1, 1), lambda i: (0, 0)),
            pl.BlockSpec((1, 1), lambda i: (0, 0)),
            pl.BlockSpec((CBLK, 2 * DIM_V), lambda i: (i, 0)),
            pl.BlockSpec((CBLK, NUM_WORDS), lambda i: (i, 0)),
            pl.BlockSpec((1, N_CLUSTERS), lambda i: (0, 0)),
        ],
        out_specs=(
            pl.BlockSpec((NPAIR, 1), lambda i: (0, 0)),
            pl.BlockSpec((TOPK, B), lambda i: (0, 0)),
            pl.BlockSpec((B * C_FM, 1), lambda i: (0, 0)),
        ),
        out_shape=(
            jax.ShapeDtypeStruct((NPAIR, 1), jnp.int32),
            jax.ShapeDtypeStruct((TOPK, B), jnp.int32),
            jax.ShapeDtypeStruct((B * C_FM, 1), f32),
        ),
        scratch_shapes=[
            pltpu.VMEM((NPAIR, TOP_POS), f32),
            pltpu.VMEM((NPAIR, TOP_POS), jnp.int32),
            pltpu.VMEM((NPW, NUM_WORDS), f32),
            pltpu.VMEM((2 * DIM_V, NPAIR), f32),
            pltpu.VMEM((NPAIR, TOP_POS), f32),
            pltpu.VMEM((NPAIR, TOP_POS), jnp.int32),
            pltpu.VMEM((NPAIR, TOP_POS), f32),
            pltpu.VMEM((NPAIR, TOP_POS), jnp.int32),
            pltpu.VMEM((NPAIR, CBLK), f32),
        ],
    )(pred_r, fs_t, xfm_r, fmax, fmin, cluster_weights, cli, vcc)

    decision_words = dwords.reshape(B, TOP_SEQ)
    decision_classes_topk = dcls.T
    feature_v = fv.reshape(B, C_FM)
    return (predicts_t, feature_v, decision_words, decision_classes_topk)
